# baseline (speedup 1.0000x reference)
import functools

import jax
import jax.numpy as jnp
from jax import lax
from jax.experimental import pallas as pl
from jax.experimental.pallas import tpu as pltpu
from jax.experimental.pallas import tpu_sc as plsc

BATCH = 16384
DIMS = (16, 24, 32)
CH = 128
L = 16


@functools.lru_cache(maxsize=None)
def _make_kernel(col: int, dim: int):
    info = plsc.get_sparse_core_info()
    nc, ns = info.num_cores, info.num_subcores
    nw = nc * ns
    b_per_w = BATCH // nw
    n_ch = b_per_w // CH
    g_per_ch = CH // L

    mesh = plsc.VectorSubcoreMesh(core_axis_name="c", subcore_axis_name="s")

    @functools.partial(
        pl.kernel,
        mesh=mesh,
        out_type=jax.ShapeDtypeStruct((BATCH, dim), jnp.float32),
        scratch_types=[
            pltpu.VMEM((b_per_w, 3), jnp.int32),
            pltpu.VMEM((n_ch, CH), jnp.int32),
            pltpu.VMEM((b_per_w, dim), jnp.float32),
            pltpu.SemaphoreType.DMA,
        ],
        compiler_params=pltpu.CompilerParams(
            use_tc_tiling_on_sc=False, needs_layout_passes=False
        ),
    )
    def branch_gather(gidx, tbl, out, raw_v, idx_v, rows_v, sem):
        wid = lax.axis_index("s") * nc + lax.axis_index("c")
        base = wid * b_per_w
        pltpu.sync_copy(gidx.at[pl.ds(base, b_per_w)], raw_v)
        lane = lax.broadcasted_iota(jnp.int32, (L,), 0)
        cols = jnp.full((L,), col, jnp.int32)
        for c in range(n_ch):
            for g in range(g_per_ch):
                rows = lane + (c * CH + g * L)
                vals = plsc.load_gather(raw_v, [rows, cols])
                idx_v[c, pl.ds(g * L, L)] = vals
        copies = [
            pltpu.async_copy(
                tbl.at[idx_v.at[c]], rows_v.at[pl.ds(c * CH, CH)], sem
            )
            for c in range(n_ch)
        ]
        for cp in copies:
            cp.wait()
        pltpu.sync_copy(rows_v, out.at[pl.ds(base, b_per_w)])

    return branch_gather


def kernel(grid_idx, E0, E1, E2):
    gidx = grid_idx.astype(jnp.int32)
    outs = [
        _make_kernel(j, d)(gidx, tbl)
        for j, (tbl, d) in enumerate(((E0, DIMS[0]), (E1, DIMS[1]), (E2, DIMS[2])))
    ]
    return jnp.concatenate(outs, axis=1)


# speedup vs baseline: 2.8583x; 2.8583x over previous
import functools

import jax
import jax.numpy as jnp
from jax import lax
from jax.experimental import pallas as pl
from jax.experimental.pallas import tpu as pltpu
from jax.experimental.pallas import tpu_sc as plsc

V = 100001
B = 16384
HB = B // 2
GRP = 16


@functools.lru_cache(maxsize=None)
def _make_kernel():
    info = plsc.get_sparse_core_info()
    nc, ns = info.num_cores, info.num_subcores
    mesh = plsc.VectorSubcoreMesh(core_axis_name="c", subcore_axis_name="s")

    @functools.partial(
        pl.kernel,
        mesh=mesh,
        out_type=jax.ShapeDtypeStruct((72, B), jnp.float32),
        scratch_types=[
            pltpu.VMEM((1, V), jnp.float32),
            pltpu.VMEM((1, HB), jnp.int32),
            pltpu.VMEM((1, HB), jnp.float32),
        ],
        compiler_params=pltpu.CompilerParams(
            use_tc_tiling_on_sc=True, needs_layout_passes=False
        ),
    )
    def plane_gather(e0t, e1t, e2t, gidxt, out, plane_v, idx_v, res_v):
        wid = lax.axis_index("s") * nc + lax.axis_index("c")

        def do_plane(tbl, j, dd, orow):
            pltpu.sync_copy(tbl.at[pl.ds(dd, 1)], plane_v)
            for h in range(2):
                pltpu.sync_copy(gidxt.at[pl.ds(j, 1), pl.ds(h * HB, HB)], idx_v)

                def body(i, carry):
                    o = i * GRP
                    idx16 = idx_v[0, pl.ds(o, GRP)]
                    zero16 = jnp.zeros((GRP,), jnp.int32)
                    res_v[0, pl.ds(o, GRP)] = plsc.load_gather(
                        plane_v, [zero16, idx16]
                    )
                    return carry

                lax.fori_loop(0, HB // GRP, body, (), unroll=8)
                pltpu.sync_copy(res_v, out.at[pl.ds(orow, 1), pl.ds(h * HB, HB)])

        @pl.when(wid < 16)
        def _():
            do_plane(e0t, 0, wid, wid)

        @pl.when(wid >= 16)
        def _():
            do_plane(e1t, 1, wid - 16, wid)

        @pl.when(wid < 8)
        def _():
            do_plane(e1t, 1, wid + 16, wid + 32)

        @pl.when(wid >= 8)
        def _():
            do_plane(e2t, 2, wid - 8, wid + 32)

        @pl.when(wid < 8)
        def _():
            do_plane(e2t, 2, wid + 24, wid + 64)

    def run(grid_idx, e0, e1, e2):
        gidxt = grid_idx.astype(jnp.int32).T
        out_t = plane_gather(e0.T, e1.T, e2.T, gidxt)
        return out_t.T

    return run


def kernel(grid_idx, E0, E1, E2):
    return _make_kernel()(grid_idx, E0, E1, E2)


# speedup vs baseline: 4.1548x; 1.4536x over previous
import functools

import jax
import jax.numpy as jnp
from jax import lax
from jax.experimental import pallas as pl
from jax.experimental.pallas import tpu as pltpu
from jax.experimental.pallas import tpu_sc as plsc

V = 100001
B = 16384
HB = B // 2
GRP = 16


@functools.lru_cache(maxsize=None)
def _make_kernel():
    info = plsc.get_sparse_core_info()
    nc, ns = info.num_cores, info.num_subcores
    mesh = plsc.VectorSubcoreMesh(core_axis_name="c", subcore_axis_name="s")

    @functools.partial(
        pl.kernel,
        mesh=mesh,
        out_type=jax.ShapeDtypeStruct((72, B), jnp.float32),
        scratch_types=[
            pltpu.VMEM((1, V), jnp.float32),
            pltpu.VMEM((1, HB), jnp.int32),
            pltpu.VMEM((1, HB), jnp.float32),
        ],
        compiler_params=pltpu.CompilerParams(
            use_tc_tiling_on_sc=True, needs_layout_passes=False
        ),
    )
    def plane_gather(e0t, e1t, e2t, gidxt, out, plane_v, idx_v, res_v):
        wid = lax.axis_index("s") * nc + lax.axis_index("c")

        def do_plane(tbl, j, dd, orow):
            pltpu.sync_copy(tbl.at[pl.ds(dd, 1)], plane_v)
            zero16 = jnp.zeros((GRP,), jnp.int32)
            for h in range(2):
                pltpu.sync_copy(gidxt.at[pl.ds(j, 1), pl.ds(h * HB, HB)], idx_v)

                @plsc.parallel_loop(0, HB, step=GRP, unroll=8)
                def body(o):
                    idx16 = idx_v[0, pl.ds(o, GRP)]
                    res_v[0, pl.ds(o, GRP)] = plsc.load_gather(
                        plane_v, [zero16, idx16]
                    )

                pltpu.sync_copy(res_v, out.at[pl.ds(orow, 1), pl.ds(h * HB, HB)])

        @pl.when(wid < 16)
        def _():
            do_plane(e0t, 0, wid, wid)

        @pl.when(wid >= 16)
        def _():
            do_plane(e1t, 1, wid - 16, wid)

        @pl.when(wid < 8)
        def _():
            do_plane(e1t, 1, wid + 16, wid + 32)

        @pl.when(wid >= 8)
        def _():
            do_plane(e2t, 2, wid - 8, wid + 32)

        @pl.when(wid < 8)
        def _():
            do_plane(e2t, 2, wid + 24, wid + 64)

    def run(grid_idx, e0, e1, e2):
        gidxt = grid_idx.astype(jnp.int32).T
        out_t = plane_gather(e0.T, e1.T, e2.T, gidxt)
        return out_t.T

    return run


def kernel(grid_idx, E0, E1, E2):
    return _make_kernel()(grid_idx, E0, E1, E2)
